# Initial kernel scaffold; baseline (speedup 1.0000x reference)
import jax
import jax.numpy as jnp
from jax.experimental import pallas as pl


def kernel(hidden_states, gate_weight, w_gate, w_up, w_down, sh_gate, sh_up, sh_down):
    raise NotImplementedError("write your pallas kernel here")


# speedup vs baseline: 2.3209x; 2.3209x over previous
import functools
import jax
import jax.numpy as jnp
from jax import lax
from jax.experimental import pallas as pl
from jax.experimental.pallas import tpu as pltpu
from jax.experimental.pallas import tpu_sc as plsc

E = 64
TOPK = 2
D = 2048
DFF = 1408
DSH = 2816
S = 2048

BM = 128
T = (S * TOPK) // BM
WU = T + E - 1
BF = 128
KG = DFF // BF
KS = DSH // BF
TS = S // BM


def _router_body(x_ref, gw_ref, w0_ref, w1_ref, e0_ref, e1_ref):
    x = x_ref[...]
    logits = lax.dot_general(x, gw_ref[...], (((1,), (1,)), ((), ())),
                             preferred_element_type=jnp.float32)
    m = jnp.max(logits, axis=1, keepdims=True)
    p = jnp.exp(logits - m)
    z = jnp.sum(p, axis=1, keepdims=True)
    iota = lax.broadcasted_iota(jnp.int32, p.shape, 1)
    v0 = jnp.max(p, axis=1, keepdims=True)
    i0 = jnp.min(jnp.where(p == v0, iota, E), axis=1, keepdims=True)
    p2 = jnp.where(iota == i0, -1.0, p)
    v1 = jnp.max(p2, axis=1, keepdims=True)
    i1 = jnp.min(jnp.where(p2 == v1, iota, E), axis=1, keepdims=True)
    s0 = v0 / z
    s1 = v1 / z
    denom = s0 + s1 + 1e-20
    w0_ref[...] = s0 / denom
    w1_ref[...] = s1 / denom
    e0_ref[...] = i0
    e1_ref[...] = i1


def _run_router(x, gate_weight):
    return pl.pallas_call(
        _router_body,
        grid=(TS,),
        in_specs=[
            pl.BlockSpec((BM, D), lambda t: (t, 0)),
            pl.BlockSpec((E, D), lambda t: (0, 0)),
        ],
        out_specs=[
            pl.BlockSpec((BM, 1), lambda t: (t, 0)),
            pl.BlockSpec((BM, 1), lambda t: (t, 0)),
            pl.BlockSpec((BM, 1), lambda t: (t, 0)),
            pl.BlockSpec((BM, 1), lambda t: (t, 0)),
        ],
        out_shape=[
            jax.ShapeDtypeStruct((S, 1), jnp.float32),
            jax.ShapeDtypeStruct((S, 1), jnp.float32),
            jax.ShapeDtypeStruct((S, 1), jnp.int32),
            jax.ShapeDtypeStruct((S, 1), jnp.int32),
        ],
    )(x, gate_weight)


def _make_sc_gather(V, B):
    info = plsc.get_sparse_core_info()
    NW = info.num_cores * info.num_subcores
    b_per_w = B // NW
    CH = 32
    n_ch = b_per_w // CH
    mesh = plsc.VectorSubcoreMesh(core_axis_name="c", subcore_axis_name="s")

    @functools.partial(
        pl.kernel, mesh=mesh,
        out_type=jax.ShapeDtypeStruct((B, D), jnp.float32),
        scratch_types=[
            pltpu.VMEM((CH,), jnp.int32),
            pltpu.VMEM((CH, D), jnp.float32),
            pltpu.SemaphoreType.DMA,
        ],
    )
    def k(table_hbm, idx_hbm, out_hbm, idx_v, rows_v, sem):
        wid = lax.axis_index("s") * info.num_cores + lax.axis_index("c")
        base = wid * b_per_w
        for c in range(n_ch):
            off = base + c * CH
            pltpu.sync_copy(idx_hbm.at[pl.ds(off, CH)], idx_v)
            pltpu.async_copy(table_hbm.at[idx_v], rows_v, sem).wait()
            pltpu.sync_copy(rows_v, out_hbm.at[pl.ds(off, CH)])

    return k


def _group_body(ue_ref, ut_ref, us_ref, uen_ref, uf_ref,
                xs_ref, sw_ref, wg_ref, wuq_ref, wd_ref, y_ref):
    u = pl.program_id(0)
    k = pl.program_id(1)
    x = xs_ref[...]
    g = wg_ref[0]
    up = wuq_ref[0]
    hg = lax.dot_general(x, g, (((1,), (1,)), ((), ())),
                         preferred_element_type=jnp.float32)
    hu = lax.dot_general(x, up, (((1,), (1,)), ((), ())),
                         preferred_element_type=jnp.float32)
    h = hg * jax.nn.sigmoid(hg) * hu
    r = lax.broadcasted_iota(jnp.int32, (BM, 1), 0)
    valid = (r >= us_ref[u]) & (r < uen_ref[u])
    h = h * jnp.where(valid, sw_ref[...], 0.0)
    d = wd_ref[0]
    part = lax.dot_general(h, d, (((1,), (1,)), ((), ())),
                           preferred_element_type=jnp.float32)
    init = (uf_ref[u] != 0) & (k == 0)

    @pl.when(init)
    def _():
        y_ref[...] = part

    @pl.when(jnp.logical_not(init))
    def _():
        y_ref[...] += part


def _run_grouped(xs, sw, w_gate, w_up, w_down, ue, ut, us, uen, uf):
    grid_spec = pltpu.PrefetchScalarGridSpec(
        num_scalar_prefetch=5,
        grid=(WU, KG),
        in_specs=[
            pl.BlockSpec((BM, D), lambda u, k, ue, ut, us, uen, uf: (ut[u], 0)),
            pl.BlockSpec((BM, 1), lambda u, k, ue, ut, us, uen, uf: (ut[u], 0)),
            pl.BlockSpec((1, BF, D),
                         lambda u, k, ue, ut, us, uen, uf: (ue[u], k, 0)),
            pl.BlockSpec((1, BF, D),
                         lambda u, k, ue, ut, us, uen, uf: (ue[u], k, 0)),
            pl.BlockSpec((1, D, BF),
                         lambda u, k, ue, ut, us, uen, uf: (ue[u], 0, k)),
        ],
        out_specs=pl.BlockSpec((BM, D), lambda u, k, ue, ut, us, uen, uf: (ut[u], 0)),
    )
    return pl.pallas_call(
        _group_body,
        grid_spec=grid_spec,
        out_shape=jax.ShapeDtypeStruct((S * TOPK, D), jnp.float32),
    )(ue, ut, us, uen, uf, xs, sw, w_gate, w_up, w_down)


def _shared_body(x_ref, g_ref, u_ref, d_ref, z0_ref, z1_ref, out_ref):
    k = pl.program_id(1)
    x = x_ref[...]
    hg = lax.dot_general(x, g_ref[...], (((1,), (1,)), ((), ())),
                         preferred_element_type=jnp.float32)
    hu = lax.dot_general(x, u_ref[...], (((1,), (1,)), ((), ())),
                         preferred_element_type=jnp.float32)
    h = hg * jax.nn.sigmoid(hg) * hu
    part = lax.dot_general(h, d_ref[...], (((1,), (1,)), ((), ())),
                           preferred_element_type=jnp.float32)

    @pl.when(k == 0)
    def _():
        out_ref[...] = part

    @pl.when(k != 0)
    def _():
        out_ref[...] += part

    @pl.when(k == KS - 1)
    def _():
        out_ref[...] += z0_ref[...] + z1_ref[...]


def _run_shared_combine(x, sh_gate, sh_up, sh_down, z):
    return pl.pallas_call(
        _shared_body,
        grid=(TS, KS),
        in_specs=[
            pl.BlockSpec((BM, D), lambda t, k: (t, 0)),
            pl.BlockSpec((BF, D), lambda t, k: (k, 0)),
            pl.BlockSpec((BF, D), lambda t, k: (k, 0)),
            pl.BlockSpec((D, BF), lambda t, k: (0, k)),
            pl.BlockSpec((BM, D), lambda t, k: (t, 0)),
            pl.BlockSpec((BM, D), lambda t, k: (TS + t, 0)),
        ],
        out_specs=pl.BlockSpec((BM, D), lambda t, k: (t, 0)),
        out_shape=jax.ShapeDtypeStruct((S, D), jnp.float32),
    )(x, sh_gate, sh_up, sh_down, z, z)


def kernel(hidden_states, gate_weight, w_gate, w_up, w_down,
           sh_gate, sh_up, sh_down):
    bsz, seq, h = hidden_states.shape
    x = hidden_states.reshape(-1, h)

    w0, w1, e0, e1 = _run_router(x, gate_weight)

    eflat = jnp.concatenate([e0, e1], axis=1).reshape(-1)
    wflat = jnp.concatenate([w0, w1], axis=1).reshape(-1)
    perm = jnp.argsort(eflat).astype(jnp.int32)
    sorted_tid = (perm // TOPK).astype(jnp.int32)
    sorted_w = wflat[perm].reshape(S * TOPK, 1)
    pos = jnp.zeros((S * TOPK,), jnp.int32).at[perm].set(
        jnp.arange(S * TOPK, dtype=jnp.int32))
    pp = pos.reshape(S, TOPK).T.reshape(-1)

    counts = jnp.zeros((E,), jnp.int32).at[eflat].add(1)
    offs = jnp.concatenate([jnp.zeros((1,), jnp.int32),
                            jnp.cumsum(counts)[:-1].astype(jnp.int32)])
    start_tile = offs // BM
    end_tile = (offs + counts - 1) // BM
    nunits = jnp.where(counts > 0, end_tile - start_tile + 1, 0)
    cum = jnp.cumsum(nunits).astype(jnp.int32)
    total = cum[-1]
    uarr = jnp.arange(WU, dtype=jnp.int32)
    eu = jnp.searchsorted(cum, uarr, side="right").astype(jnp.int32)
    eu_c = jnp.minimum(eu, E - 1)
    prev = cum[eu_c] - nunits[eu_c]
    tile_u = start_tile[eu_c] + (uarr - prev)
    valid_u = uarr < total
    last_e = eflat[perm[-1]]
    ue = jnp.where(valid_u, eu_c, last_e).astype(jnp.int32)
    ut = jnp.where(valid_u, tile_u, T - 1).astype(jnp.int32)
    us = jnp.where(valid_u,
                   jnp.maximum(offs[eu_c] - tile_u * BM, 0), 0).astype(jnp.int32)
    uen = jnp.where(valid_u,
                    jnp.minimum(offs[eu_c] + counts[eu_c] - tile_u * BM, BM),
                    0).astype(jnp.int32)
    uf = jnp.where(valid_u,
                   jnp.concatenate([jnp.ones((1,), jnp.int32),
                                    (ut[1:] != ut[:-1]).astype(jnp.int32)]),
                   0).astype(jnp.int32)

    xs = _make_sc_gather(S, S * TOPK)(x, sorted_tid)

    y = _run_grouped(xs, sorted_w, w_gate, w_up, w_down, ue, ut, us, uen, uf)

    z = _make_sc_gather(S * TOPK, S * TOPK)(y, pp)

    out = _run_shared_combine(x, sh_gate, sh_up, sh_down, z)
    return out.reshape(bsz, seq, h)
